# initial kernel scaffold (unmeasured)
import jax
import jax.numpy as jnp
from jax import lax
from jax.experimental import pallas as pl
from jax.experimental.pallas import tpu as pltpu

X_DEV = 2
CHUNK = 256


def kernel(partial, gamma):
    _, m_total, d = partial.shape
    m_blk = m_total // X_DEV
    p2d = partial.reshape(m_total, d)
    g2d = gamma.reshape(1, d)

    def body(p_ref, g_ref, o_ref, send_ref, recv_ref, send_sem, recv_sem):
        my_x = lax.axis_index("x")
        my_y = lax.axis_index("y")
        my_z = lax.axis_index("z")
        peer_x = 1 - my_x

        n_chunks = m_blk // CHUNK

        def stage(i, carry):
            send_ref[pl.ds(i * CHUNK, CHUNK), :] = p_ref[
                pl.ds(peer_x * m_blk + i * CHUNK, CHUNK), :
            ].astype(jnp.bfloat16)
            return carry

        lax.fori_loop(0, n_chunks, stage, 0)

        barrier_sem = pltpu.get_barrier_semaphore()
        pl.semaphore_signal(
            barrier_sem,
            inc=1,
            device_id=(peer_x, my_y, my_z),
            device_id_type=pl.DeviceIdType.MESH,
        )
        pl.semaphore_wait(barrier_sem, 1)

        rdma = pltpu.make_async_remote_copy(
            src_ref=send_ref,
            dst_ref=recv_ref,
            send_sem=send_sem,
            recv_sem=recv_sem,
            device_id=(peer_x, my_y, my_z),
            device_id_type=pl.DeviceIdType.MESH,
        )
        rdma.start()
        rdma.wait()

        g = g_ref[...]

        def comp(i, carry):
            mine = p_ref[pl.ds(my_x * m_blk + i * CHUNK, CHUNK), :]
            y = mine + recv_ref[pl.ds(i * CHUNK, CHUNK), :].astype(jnp.float32)
            ms = jnp.mean(y * y, axis=-1, keepdims=True)
            o_ref[pl.ds(i * CHUNK, CHUNK), :] = y * lax.rsqrt(ms + 1e-6) * g
            return carry

        lax.fori_loop(0, n_chunks, comp, 0)

    return pl.pallas_call(
        body,
        out_shape=jax.ShapeDtypeStruct((m_blk, d), jnp.float32),
        in_specs=[
            pl.BlockSpec(memory_space=pltpu.VMEM),
            pl.BlockSpec(memory_space=pltpu.VMEM),
        ],
        out_specs=pl.BlockSpec(memory_space=pltpu.VMEM),
        scratch_shapes=[
            pltpu.VMEM((m_blk, d), jnp.bfloat16),
            pltpu.VMEM((m_blk, d), jnp.bfloat16),
            pltpu.SemaphoreType.DMA,
            pltpu.SemaphoreType.DMA,
        ],
        compiler_params=pltpu.CompilerParams(
            collective_id=0,
            vmem_limit_bytes=110 * 1024 * 1024,
        ),
    )(p2d, g2d)


# baseline (device time: 132953 ns/iter reference)
import jax
import jax.numpy as jnp
from jax import lax
from jax.experimental import pallas as pl
from jax.experimental.pallas import tpu as pltpu

X_DEV = 2
CHUNK = 256


def kernel(partial, gamma):
    _, m_total, d = partial.shape
    m_blk = m_total // X_DEV
    p2d = partial.reshape(m_total, d)
    g2d = gamma.reshape(1, d)

    def body(
        p_ref, g_ref, o_ref, send_ref, recv_ref, obuf_ref, send_sem, recv_sem, out_sem
    ):
        my_x = lax.axis_index("x")
        my_y = lax.axis_index("y")
        my_z = lax.axis_index("z")
        peer_x = 1 - my_x

        n_chunks = m_blk // CHUNK

        def stage(i, carry):
            send_ref[pl.ds(i * CHUNK, CHUNK), :] = p_ref[
                pl.ds(peer_x * m_blk + i * CHUNK, CHUNK), :
            ].astype(jnp.bfloat16)
            return carry

        lax.fori_loop(0, n_chunks, stage, 0)

        barrier_sem = pltpu.get_barrier_semaphore()
        pl.semaphore_signal(
            barrier_sem,
            inc=1,
            device_id=(peer_x, my_y, my_z),
            device_id_type=pl.DeviceIdType.MESH,
        )
        pl.semaphore_wait(barrier_sem, 1)

        rdma = pltpu.make_async_remote_copy(
            src_ref=send_ref,
            dst_ref=recv_ref,
            send_sem=send_sem,
            recv_sem=recv_sem,
            device_id=(peer_x, my_y, my_z),
            device_id_type=pl.DeviceIdType.MESH,
        )
        rdma.start()
        rdma.wait()

        g = g_ref[...]

        def comp(i, carry):
            mine = p_ref[pl.ds(my_x * m_blk + i * CHUNK, CHUNK), :]
            y = mine + recv_ref[pl.ds(i * CHUNK, CHUNK), :].astype(jnp.float32)
            ms = jnp.mean(y * y, axis=-1, keepdims=True)
            obuf_ref[...] = y * lax.rsqrt(ms + 1e-6) * g
            copy = pltpu.make_async_copy(
                obuf_ref, o_ref.at[pl.ds(i * CHUNK, CHUNK), :], out_sem
            )
            copy.start()
            copy.wait()
            return carry

        lax.fori_loop(0, n_chunks, comp, 0)

    return pl.pallas_call(
        body,
        out_shape=jax.ShapeDtypeStruct((m_blk, d), jnp.float32),
        in_specs=[
            pl.BlockSpec(memory_space=pltpu.VMEM),
            pl.BlockSpec(memory_space=pltpu.VMEM),
        ],
        out_specs=pl.BlockSpec(memory_space=pl.ANY),
        scratch_shapes=[
            pltpu.VMEM((m_blk, d), jnp.bfloat16),
            pltpu.VMEM((m_blk, d), jnp.bfloat16),
            pltpu.VMEM((CHUNK, d), jnp.float32),
            pltpu.SemaphoreType.DMA,
            pltpu.SemaphoreType.DMA,
            pltpu.SemaphoreType.DMA,
        ],
        compiler_params=pltpu.CompilerParams(
            collective_id=0,
            vmem_limit_bytes=110 * 1024 * 1024,
        ),
    )(p2d, g2d)


# device time: 84697 ns/iter; 1.5697x vs baseline; 1.5697x over previous
import jax
import jax.numpy as jnp
from jax import lax
from jax.experimental import pallas as pl
from jax.experimental.pallas import tpu as pltpu

X_DEV = 2
CHUNK = 128


def kernel(partial, gamma):
    _, m_total, d = partial.shape
    m_blk = m_total // X_DEV
    m_half = m_blk // 2
    n_ch = m_half // CHUNK
    p2d = partial.reshape(m_total, d)
    g2d = gamma.reshape(1, d)

    def body(
        p_ref,
        g_ref,
        o_ref,
        xsend_ref,
        recv_ref,
        obuf_ref,
        xsend_sems,
        xrecv_sems,
        zsend_sems,
        zrecv_sems,
        out_sem,
    ):
        my_x = lax.axis_index("x")
        my_y = lax.axis_index("y")
        my_z = lax.axis_index("z")
        peer_x = 1 - my_x
        q = lax.rem(my_z, 2)
        pz = my_z - 2 * q + 1

        barrier_sem = pltpu.get_barrier_semaphore()
        pl.semaphore_signal(
            barrier_sem,
            inc=1,
            device_id=(peer_x, my_y, my_z),
            device_id_type=pl.DeviceIdType.MESH,
        )
        pl.semaphore_signal(
            barrier_sem,
            inc=1,
            device_id=(my_x, my_y, pz),
            device_id_type=pl.DeviceIdType.MESH,
        )
        pl.semaphore_wait(barrier_sem, 2)

        def x_rdma(c):
            off = q * m_half + c * CHUNK
            return pltpu.make_async_remote_copy(
                src_ref=xsend_ref.at[pl.ds(c * CHUNK, CHUNK), :],
                dst_ref=recv_ref.at[pl.ds(off, CHUNK), :],
                send_sem=xsend_sems.at[c],
                recv_sem=xrecv_sems.at[c],
                device_id=(peer_x, my_y, my_z),
                device_id_type=pl.DeviceIdType.MESH,
            )

        def z_rdma(c):
            off = q * m_half + c * CHUNK
            return pltpu.make_async_remote_copy(
                src_ref=recv_ref.at[pl.ds(off, CHUNK), :],
                dst_ref=recv_ref.at[pl.ds(off, CHUNK), :],
                send_sem=zsend_sems.at[c],
                recv_sem=zrecv_sems.at[c],
                device_id=(my_x, my_y, pz),
                device_id_type=pl.DeviceIdType.MESH,
            )

        def z_recv_descriptor(c):
            off = (1 - q) * m_half + c * CHUNK
            return pltpu.make_async_remote_copy(
                src_ref=recv_ref.at[pl.ds(off, CHUNK), :],
                dst_ref=recv_ref.at[pl.ds(off, CHUNK), :],
                send_sem=zsend_sems.at[c],
                recv_sem=zrecv_sems.at[c],
                device_id=(my_x, my_y, pz),
                device_id_type=pl.DeviceIdType.MESH,
            )

        for c in range(n_ch):
            src_row = peer_x * m_blk + q * m_half + c * CHUNK
            xsend_ref[pl.ds(c * CHUNK, CHUNK), :] = p_ref[
                pl.ds(src_row, CHUNK), :
            ].astype(jnp.bfloat16)
            x_rdma(c).start()

        g = g_ref[...]

        def compute_chunk(blk_off):
            mine = p_ref[pl.ds(my_x * m_blk + blk_off, CHUNK), :]
            y = mine + recv_ref[pl.ds(blk_off, CHUNK), :].astype(jnp.float32)
            ms = jnp.mean(y * y, axis=-1, keepdims=True)
            obuf_ref[...] = y * lax.rsqrt(ms + 1e-6) * g
            copy = pltpu.make_async_copy(
                obuf_ref, o_ref.at[pl.ds(blk_off, CHUNK), :], out_sem
            )
            copy.start()
            copy.wait()

        for c in range(n_ch):
            x_rdma(c).wait_recv()
            z_rdma(c).start()
            compute_chunk(q * m_half + c * CHUNK)

        for c in range(n_ch):
            z_recv_descriptor(c).wait_recv()
            compute_chunk((1 - q) * m_half + c * CHUNK)

        for c in range(n_ch):
            x_rdma(c).wait_send()
            z_rdma(c).wait_send()

    return pl.pallas_call(
        body,
        out_shape=jax.ShapeDtypeStruct((m_blk, d), jnp.float32),
        in_specs=[
            pl.BlockSpec(memory_space=pltpu.VMEM),
            pl.BlockSpec(memory_space=pltpu.VMEM),
        ],
        out_specs=pl.BlockSpec(memory_space=pl.ANY),
        scratch_shapes=[
            pltpu.VMEM((m_half, d), jnp.bfloat16),
            pltpu.VMEM((m_blk, d), jnp.bfloat16),
            pltpu.VMEM((CHUNK, d), jnp.float32),
            pltpu.SemaphoreType.DMA((m_half // CHUNK,)),
            pltpu.SemaphoreType.DMA((m_half // CHUNK,)),
            pltpu.SemaphoreType.DMA((m_half // CHUNK,)),
            pltpu.SemaphoreType.DMA((m_half // CHUNK,)),
            pltpu.SemaphoreType.DMA,
        ],
        compiler_params=pltpu.CompilerParams(
            collective_id=0,
            vmem_limit_bytes=60 * 1024 * 1024,
        ),
    )(p2d, g2d)


# device time: 76460 ns/iter; 1.7389x vs baseline; 1.1077x over previous
import jax
import jax.numpy as jnp
from jax import lax
from jax.experimental import pallas as pl
from jax.experimental.pallas import tpu as pltpu

X_DEV = 2
CHUNK = 128


def kernel(partial, gamma):
    _, m_total, d = partial.shape
    m_blk = m_total // X_DEV
    m_half = m_blk // 2
    n_ch = m_half // CHUNK
    p2d = partial.reshape(m_total, d)
    g2d = gamma.reshape(1, d)

    def body(
        p_ref,
        g_ref,
        o_ref,
        xsend_ref,
        recv_ref,
        pin_ref,
        mine_ref,
        obuf_ref,
        xsend_sems,
        xrecv_sems,
        zsend_sems,
        zrecv_sems,
        pin_sems,
        mine_sems,
        out_sem,
    ):
        my_x = lax.axis_index("x")
        my_y = lax.axis_index("y")
        my_z = lax.axis_index("z")
        peer_x = 1 - my_x
        q = lax.rem(my_z, 2)
        pz = my_z - 2 * q + 1

        barrier_sem = pltpu.get_barrier_semaphore()
        pl.semaphore_signal(
            barrier_sem,
            inc=1,
            device_id=(peer_x, my_y, my_z),
            device_id_type=pl.DeviceIdType.MESH,
        )
        pl.semaphore_signal(
            barrier_sem,
            inc=1,
            device_id=(my_x, my_y, pz),
            device_id_type=pl.DeviceIdType.MESH,
        )
        pl.semaphore_wait(barrier_sem, 2)

        def x_rdma(c):
            off = q * m_half + c * CHUNK
            return pltpu.make_async_remote_copy(
                src_ref=xsend_ref.at[pl.ds(c * CHUNK, CHUNK), :],
                dst_ref=recv_ref.at[pl.ds(off, CHUNK), :],
                send_sem=xsend_sems.at[c],
                recv_sem=xrecv_sems.at[c],
                device_id=(peer_x, my_y, my_z),
                device_id_type=pl.DeviceIdType.MESH,
            )

        def z_rdma(c):
            off = q * m_half + c * CHUNK
            return pltpu.make_async_remote_copy(
                src_ref=recv_ref.at[pl.ds(off, CHUNK), :],
                dst_ref=recv_ref.at[pl.ds(off, CHUNK), :],
                send_sem=zsend_sems.at[c],
                recv_sem=zrecv_sems.at[c],
                device_id=(my_x, my_y, pz),
                device_id_type=pl.DeviceIdType.MESH,
            )

        def z_recv_descriptor(c):
            off = (1 - q) * m_half + c * CHUNK
            return pltpu.make_async_remote_copy(
                src_ref=recv_ref.at[pl.ds(off, CHUNK), :],
                dst_ref=recv_ref.at[pl.ds(off, CHUNK), :],
                send_sem=zsend_sems.at[c],
                recv_sem=zrecv_sems.at[c],
                device_id=(my_x, my_y, pz),
                device_id_type=pl.DeviceIdType.MESH,
            )

        def stage_load(c):
            src_row = peer_x * m_blk + q * m_half + c * CHUNK
            return pltpu.make_async_copy(
                p_ref.at[pl.ds(src_row, CHUNK), :],
                pin_ref.at[c % 2],
                pin_sems.at[c % 2],
            )

        def blk_off(i):
            if i < n_ch:
                return q * m_half + i * CHUNK
            return (1 - q) * m_half + (i - n_ch) * CHUNK

        def mine_load(i):
            return pltpu.make_async_copy(
                p_ref.at[pl.ds(my_x * m_blk + blk_off(i), CHUNK), :],
                mine_ref.at[i % 2],
                mine_sems.at[i % 2],
            )

        stage_load(0).start()
        for c in range(n_ch):
            stage_load(c).wait()
            if c + 1 < n_ch:
                stage_load(c + 1).start()
            xsend_ref[pl.ds(c * CHUNK, CHUNK), :] = pin_ref[c % 2].astype(
                jnp.bfloat16
            )
            x_rdma(c).start()

        g = g_ref[...]
        mine_load(0).start()
        mine_load(1).start()

        def compute_chunk(i):
            off = blk_off(i)
            mine_load(i).wait()
            y = mine_ref[i % 2] + recv_ref[pl.ds(off, CHUNK), :].astype(
                jnp.float32
            )
            if i + 2 < 2 * n_ch:
                mine_load(i + 2).start()
            ms = jnp.mean(y * y, axis=-1, keepdims=True)
            obuf_ref[...] = y * lax.rsqrt(ms + 1e-6) * g
            copy = pltpu.make_async_copy(
                obuf_ref, o_ref.at[pl.ds(off, CHUNK), :], out_sem
            )
            copy.start()
            copy.wait()

        for c in range(n_ch):
            x_rdma(c).wait_recv()
            z_rdma(c).start()
            compute_chunk(c)

        for c in range(n_ch):
            z_recv_descriptor(c).wait_recv()
            compute_chunk(n_ch + c)

        for c in range(n_ch):
            x_rdma(c).wait_send()
            z_rdma(c).wait_send()

    return pl.pallas_call(
        body,
        out_shape=jax.ShapeDtypeStruct((m_blk, d), jnp.float32),
        in_specs=[
            pl.BlockSpec(memory_space=pl.ANY),
            pl.BlockSpec(memory_space=pltpu.VMEM),
        ],
        out_specs=pl.BlockSpec(memory_space=pl.ANY),
        scratch_shapes=[
            pltpu.VMEM((m_half, d), jnp.bfloat16),
            pltpu.VMEM((m_blk, d), jnp.bfloat16),
            pltpu.VMEM((2, CHUNK, d), jnp.float32),
            pltpu.VMEM((2, CHUNK, d), jnp.float32),
            pltpu.VMEM((CHUNK, d), jnp.float32),
            pltpu.SemaphoreType.DMA((m_half // CHUNK,)),
            pltpu.SemaphoreType.DMA((m_half // CHUNK,)),
            pltpu.SemaphoreType.DMA((m_half // CHUNK,)),
            pltpu.SemaphoreType.DMA((m_half // CHUNK,)),
            pltpu.SemaphoreType.DMA((2,)),
            pltpu.SemaphoreType.DMA((2,)),
            pltpu.SemaphoreType.DMA,
        ],
        compiler_params=pltpu.CompilerParams(
            collective_id=0,
            vmem_limit_bytes=48 * 1024 * 1024,
        ),
    )(p2d, g2d)
